# baseline (speedup 1.0000x reference)
import functools

import jax
import jax.numpy as jnp
from jax import lax
from jax.experimental import pallas as pl
from jax.experimental.pallas import tpu as pltpu
from jax.experimental.pallas import tpu_sc as plsc

L = 16
CHUNK = 128


def _make_kernel(B, K, n_workers):
    bpw = B // n_workers
    n_chunks = bpw // CHUNK
    mesh = plsc.VectorSubcoreMesh(core_axis_name="c", subcore_axis_name="s")

    @functools.partial(
        pl.kernel,
        out_type=jax.ShapeDtypeStruct((B,), jnp.float32),
        mesh=mesh,
        compiler_params=pltpu.CompilerParams(
            needs_layout_passes=False, use_tc_tiling_on_sc=False),
        scratch_types=[
            pltpu.VMEM((n_chunks, CHUNK), jnp.int32),
            pltpu.VMEM((n_chunks, CHUNK), jnp.int32),
            pltpu.VMEM((bpw, K), jnp.float32),
            pltpu.VMEM((bpw, K), jnp.float32),
            pltpu.VMEM((bpw,), jnp.float32),
            pltpu.SemaphoreType.DMA,
        ],
    )
    def deep_mf(pu_hbm, qi_hbm, users_hbm, items_hbm, out_hbm,
                uidx_v, iidx_v, urows_v, irows_v, out_v, sem):
        wid = lax.axis_index("s") * 2 + lax.axis_index("c")
        chunk_base = wid * n_chunks

        pltpu.sync_copy(users_hbm.at[pl.ds(chunk_base, n_chunks)], uidx_v)
        pltpu.sync_copy(items_hbm.at[pl.ds(chunk_base, n_chunks)], iidx_v)

        copies = []
        for j in range(n_chunks):
            copies.append(pltpu.async_copy(
                pu_hbm.at[uidx_v.at[j]],
                urows_v.at[pl.ds(j * CHUNK, CHUNK)], sem))
            copies.append(pltpu.async_copy(
                qi_hbm.at[iidx_v.at[j]],
                irows_v.at[pl.ds(j * CHUNK, CHUNK)], sem))
        for c in copies:
            c.wait()

        lane = lax.iota(jnp.int32, L)

        def group_body(g, carry):
            rows = g * L + lane
            acc = jnp.zeros((L,), jnp.float32)
            for j in range(K):
                col = jnp.full((L,), j, jnp.int32)
                uj = plsc.load_gather(urows_v, [rows, col])
                vj = plsc.load_gather(irows_v, [rows, col])
                acc = acc + uj * vj
            out_v[pl.ds(g * L, L)] = acc
            return carry

        lax.fori_loop(0, bpw // L, group_body, 0)

        pltpu.sync_copy(out_v, out_hbm.at[pl.ds(wid * bpw, bpw)])

    return deep_mf


@jax.jit
def kernel(users, items, pu_table, qi_table):
    B = users.shape[0]
    K = pu_table.shape[1]
    n_workers = 32
    users2d = users.reshape(-1).astype(jnp.int32).reshape(-1, CHUNK)
    items2d = items.reshape(-1).astype(jnp.int32).reshape(-1, CHUNK)
    out = _make_kernel(B, K, n_workers)(pu_table, qi_table, users2d, items2d)
    return out.reshape(B, 1)
